# baseline (speedup 1.0000x reference)
import functools

import jax
import jax.numpy as jnp
from jax import lax
from jax.experimental import pallas as pl
from jax.experimental.pallas import tpu as pltpu
from jax.experimental.pallas import tpu_sc as plsc

_N = 10000
_E = 320000
_D = 128
_G = 64

_NC = 2
_NS = 16
_NW = _NC * _NS
_CHUNK = 40
_EPW = _E // _NW
_NCHUNK = _EPW // _CHUNK
_NPAD = 10240
_RPT = _NPAD // _NS

_BLK = 1000
_NBLK = _N // _BLK


_NBUF = 5
_NPH = 5
_PCH = _NCHUNK // _NPH
_PGRP = _PCH // _NBUF


def _sc_gather_scatter(x, src3, dst3):
    mesh = plsc.VectorSubcoreMesh(core_axis_name="c", subcore_axis_name="s")

    @functools.partial(
        pl.kernel,
        mesh=mesh,
        out_type=jax.ShapeDtypeStruct((_NC, _NPAD, _D), jnp.float32),
        scratch_types=[
            pltpu.VMEM_SHARED((_NPAD, _D), jnp.float32),
            pltpu.VMEM((_PCH, _CHUNK), jnp.int32),
            pltpu.VMEM((_PCH, _CHUNK), jnp.int32),
            pltpu.VMEM((_NBUF, _CHUNK, _D), jnp.float32),
        ] + [pltpu.SemaphoreType.DMA] * (2 * _NBUF),
    )
    def k(x_hbm, src_hbm, dst_hbm, out_hbm, acc, sidx, didx, rows, *sems):
        gsems = sems[:_NBUF]
        ssems = sems[_NBUF:]
        c = lax.axis_index("c")
        s = lax.axis_index("s")
        rowbase = s * _RPT
        wid = c * _NS + s

        zvec = jnp.zeros((16,), jnp.float32)

        def zrow(r, carry):
            for q in range(_D // 16):
                rows[0, r, pl.ds(q * 16, 16)] = zvec
            return carry

        lax.fori_loop(0, _CHUNK, zrow, 0)
        for t in range(_RPT // _CHUNK):
            pltpu.sync_copy(rows.at[0],
                            acc.at[pl.ds(rowbase + t * _CHUNK, _CHUNK)])
        plsc.subcore_barrier()

        def wait_g(b, j):
            pltpu.make_async_copy(x_hbm.at[sidx.at[j]], rows.at[b],
                                  gsems[b]).wait()

        def fire_g(b, j):
            pltpu.async_copy(x_hbm.at[sidx.at[j]], rows.at[b], gsems[b])

        def fire_s(b, j):
            pltpu.async_copy(rows.at[b], acc.at[didx.at[j]], ssems[b],
                             add=True)

        def wait_s(b, j):
            pltpu.make_async_copy(rows.at[b], acc.at[didx.at[j]],
                                  ssems[b]).wait()

        for p in range(_NPH):
            pltpu.sync_copy(src_hbm.at[wid, p], sidx)
            pltpu.sync_copy(dst_hbm.at[wid, p], didx)
            for b in range(_NBUF):
                fire_g(b, b)
            wait_g(0, 0)
            fire_s(0, 0)

            def body(g, carry):
                for b in range(_NBUF):
                    j = 1 + g * _NBUF + b
                    bj = (1 + b) % _NBUF
                    bp = b % _NBUF
                    wait_g(bj, j)
                    fire_s(bj, j)
                    wait_s(bp, j - 1)
                    fire_g(bp, j + _NBUF - 1)
                return carry

            lax.fori_loop(0, (_PCH - _NBUF) // _NBUF, body, 0)
            for b in range(_NBUF - 1):
                j = _PCH - _NBUF + 1 + b
                bj = j % _NBUF
                wait_g(bj, j)
                fire_s(bj, j)
            for b in range(_NBUF):
                j = _PCH - _NBUF + b
                wait_s(j % _NBUF, j)

        plsc.subcore_barrier()
        pltpu.sync_copy(acc.at[pl.ds(rowbase, _RPT)],
                        out_hbm.at[c, pl.ds(rowbase, _RPT)])

    return k(x, src3, dst3)


def _tc_finish(parts, x, gids2, W_msg, W_upd, W_cls, b2):
    def body(p_ref, x_ref, g_ref, wm_ref, wu_ref, wcls_ref, b_ref, out_ref):
        wc = jnp.dot(wm_ref[...], wu_ref[...],
                     preferred_element_type=jnp.float32)
        a = p_ref[0, :_N, :] + p_ref[1, :_N, :]
        ne = jnp.dot(a, wc, preferred_element_type=jnp.float32)
        ne = jnp.maximum(ne + x_ref[...], 0.0)
        g = g_ref[...].reshape(_N, 1)
        seg = lax.broadcasted_iota(jnp.int32, (_N, _G), 1)
        oh = (g == seg).astype(jnp.float32)
        ge = lax.dot_general(oh, ne, (((0,), (0,)), ((), ())),
                             preferred_element_type=jnp.float32)
        logits = jnp.dot(ge, wcls_ref[...],
                         preferred_element_type=jnp.float32) + b_ref[0, 0]
        out_ref[...] = 1.0 / (1.0 + jnp.exp(-logits))

    return pl.pallas_call(
        body,
        out_shape=jax.ShapeDtypeStruct((_G, 1), jnp.float32),
    )(parts, x, gids2, W_msg, W_upd, W_cls, b2)


def kernel(x, edge_index, graph_ids, W_msg, W_upd, W_cls, b_cls):
    src3 = edge_index[0].reshape(_NW, _NPH, _PCH, _CHUNK)
    dst3 = edge_index[1].reshape(_NW, _NPH, _PCH, _CHUNK)
    parts = _sc_gather_scatter(x, src3, dst3)
    gids2 = graph_ids.reshape(1, _N)
    return _tc_finish(parts, x, gids2, W_msg, W_upd, W_cls,
                      b_cls.reshape(1, 1))


# speedup vs baseline: 1.0570x; 1.0570x over previous
import functools

import jax
import jax.numpy as jnp
from jax import lax
from jax.experimental import pallas as pl
from jax.experimental.pallas import tpu as pltpu
from jax.experimental.pallas import tpu_sc as plsc

_N = 10000
_E = 320000
_D = 128
_G = 64

_NC = 2
_NS = 16
_NW = _NC * _NS
_CHUNK = 40
_EPW = _E // _NW
_NCHUNK = _EPW // _CHUNK
_NPAD = 10240
_RPT = _NPAD // _NS

_BLK = 1000
_NBLK = _N // _BLK


_NBUF = 5
_NPH = 5
_PCH = _NCHUNK // _NPH
_PGRP = _PCH // _NBUF


def _sc_gather_scatter(x, src3, dst3):
    mesh = plsc.VectorSubcoreMesh(core_axis_name="c", subcore_axis_name="s")

    @functools.partial(
        pl.kernel,
        mesh=mesh,
        out_type=jax.ShapeDtypeStruct((_NC, _NPAD, _D), jnp.float32),
        scratch_types=[
            pltpu.VMEM_SHARED((_NPAD, _D), jnp.float32),
            pltpu.VMEM((_PCH, _CHUNK), jnp.int32),
            pltpu.VMEM((_PCH, _CHUNK), jnp.int32),
            pltpu.VMEM((_NBUF, _CHUNK, _D), jnp.float32),
        ] + [pltpu.SemaphoreType.DMA] * (_NBUF + 1),
    )
    def k(x_hbm, src_hbm, dst_hbm, out_hbm, acc, sidx, didx, rows, *sems):
        isem = sems[_NBUF]
        c = lax.axis_index("c")
        s = lax.axis_index("s")
        rowbase = s * _RPT
        wid = c * _NS + s

        def fire_idx(p):
            pltpu.async_copy(src_hbm.at[wid, p], sidx, isem)
            pltpu.async_copy(dst_hbm.at[wid, p], didx, isem)

        def wait_idx(p):
            pltpu.make_async_copy(src_hbm.at[wid, p], sidx, isem).wait()
            pltpu.make_async_copy(dst_hbm.at[wid, p], didx, isem).wait()

        fire_idx(0)

        zvec = jnp.zeros((16,), jnp.float32)

        def zrow(r, carry):
            for q in range(_D // 16):
                rows[0, r, pl.ds(q * 16, 16)] = zvec
            return carry

        lax.fori_loop(0, _CHUNK, zrow, 0)
        for t in range(_RPT // _CHUNK):
            pltpu.sync_copy(rows.at[0],
                            acc.at[pl.ds(rowbase + t * _CHUNK, _CHUNK)])
        plsc.subcore_barrier()

        for p in range(_NPH):
            wait_idx(p)
            for b in range(_NBUF):
                pltpu.async_copy(x_hbm.at[sidx.at[b]], rows.at[b], sems[b])

            def body(g, carry):
                for b in range(_NBUF):
                    j = g * _NBUF + b
                    pltpu.make_async_copy(x_hbm.at[sidx.at[j]], rows.at[b],
                                          sems[b]).wait()
                    pltpu.sync_copy(rows.at[b], acc.at[didx.at[j]], add=True)
                    pltpu.async_copy(x_hbm.at[sidx.at[j + _NBUF]],
                                     rows.at[b], sems[b])
                return carry

            lax.fori_loop(0, _PGRP - 1, body, 0)
            for b in range(_NBUF):
                j = (_PGRP - 1) * _NBUF + b
                pltpu.make_async_copy(x_hbm.at[sidx.at[j]], rows.at[b],
                                      sems[b]).wait()
                pltpu.sync_copy(rows.at[b], acc.at[didx.at[j]], add=True)
            if p + 1 < _NPH:
                fire_idx(p + 1)

        plsc.subcore_barrier()
        pltpu.sync_copy(acc.at[pl.ds(rowbase, _RPT)],
                        out_hbm.at[c, pl.ds(rowbase, _RPT)])

    return k(x, src3, dst3)


def _tc_finish(parts, x, gids2, W_msg, W_upd, W_cls, b2):
    def body(p_ref, x_ref, g_ref, wm_ref, wu_ref, wcls_ref, b_ref, out_ref):
        a = p_ref[0, :_N, :] + p_ref[1, :_N, :]
        agg = jnp.dot(a, wm_ref[...], preferred_element_type=jnp.float32)
        ne = jnp.dot(agg, wu_ref[...], preferred_element_type=jnp.float32)
        ne = jnp.maximum(ne + x_ref[...], 0.0)
        g = g_ref[...].reshape(_N, 1)
        seg = lax.broadcasted_iota(jnp.int32, (_N, _G), 1)
        oh = (g == seg).astype(jnp.float32)
        ge = lax.dot_general(oh, ne, (((0,), (0,)), ((), ())),
                             preferred_element_type=jnp.float32)
        logits = jnp.dot(ge, wcls_ref[...],
                         preferred_element_type=jnp.float32) + b_ref[0, 0]
        out_ref[...] = 1.0 / (1.0 + jnp.exp(-logits))

    return pl.pallas_call(
        body,
        out_shape=jax.ShapeDtypeStruct((_G, 1), jnp.float32),
    )(parts, x, gids2, W_msg, W_upd, W_cls, b2)


def kernel(x, edge_index, graph_ids, W_msg, W_upd, W_cls, b_cls):
    src3 = edge_index[0].reshape(_NW, _NPH, _PCH, _CHUNK)
    dst3 = edge_index[1].reshape(_NW, _NPH, _PCH, _CHUNK)
    parts = _sc_gather_scatter(x, src3, dst3)
    gids2 = graph_ids.reshape(1, _N)
    return _tc_finish(parts, x, gids2, W_msg, W_upd, W_cls,
                      b_cls.reshape(1, 1))
